# Initial kernel scaffold; baseline (speedup 1.0000x reference)
import functools

import jax
import jax.numpy as jnp
from jax import lax
from jax.experimental import pallas as pl
from jax.experimental.pallas import tpu as pltpu
from jax.experimental.pallas import tpu_sc as plsc

B = 16384
L = 50
D = 16
NR = 5
N = B * L

_NC = 2
_NS = 16
_NW = _NC * _NS
_V_PER_W = N // _NW
_U_PER_W = B // _NW
_CHUNK = 2560
_NSTEP = _V_PER_W // _CHUNK

_sc_mesh = plsc.VectorSubcoreMesh(core_axis_name="c", subcore_axis_name="s")


@functools.partial(
    pl.kernel,
    out_type=[
        jax.ShapeDtypeStruct((N, D), jnp.float32),
        jax.ShapeDtypeStruct((B, D), jnp.float32),
    ],
    mesh=_sc_mesh,
    scratch_types=[
        pltpu.VMEM((_CHUNK,), jnp.int32),
        pltpu.VMEM((_CHUNK, D), jnp.float32),
        pltpu.VMEM((_U_PER_W,), jnp.int32),
        pltpu.VMEM((_U_PER_W, D), jnp.float32),
        pltpu.SemaphoreType.DMA,
    ],
)
def _sc_gather(v2e_hbm, vidx_hbm, u2e_hbm, nodes_hbm,
               euv_hbm, urep_hbm,
               idx_v, rows_v, uidx_v, urows_v, sem):
    wid = lax.axis_index("s") * _NC + lax.axis_index("c")

    ubase = pl.multiple_of(wid * _U_PER_W, 8)
    pltpu.sync_copy(nodes_hbm.at[pl.ds(ubase, _U_PER_W)], uidx_v)
    pltpu.async_copy(u2e_hbm.at[uidx_v], urows_v, sem).wait()
    pltpu.sync_copy(urows_v, urep_hbm.at[pl.ds(ubase, _U_PER_W)])

    vbase = wid * _V_PER_W

    def step(i, carry):
        base = pl.multiple_of(vbase + i * _CHUNK, 8)
        pltpu.sync_copy(vidx_hbm.at[pl.ds(base, _CHUNK)], idx_v)
        pltpu.async_copy(v2e_hbm.at[idx_v], rows_v, sem).wait()
        pltpu.sync_copy(rows_v, euv_hbm.at[pl.ds(base, _CHUNK)])
        return carry

    lax.fori_loop(0, _NSTEP, step, 0)


_BB = 64
_NB = _BB * L


def _tc_body(euv_ref, hr_ref, urep_ref, r2e_ref,
             w1a_ref, w1b_ref, b1_ref, w2_ref, b2_ref,
             a1a_ref, a1b_ref, ba1_ref, a2w_ref, ba2_ref, a3w_ref,
             out_ref):
    f32 = jnp.float32
    dot = functools.partial(jnp.dot, preferred_element_type=f32)

    euv = euv_ref[...]
    hr = hr_ref[...]
    oh = (hr == lax.broadcasted_iota(jnp.int32, (_NB, NR), 1)).astype(f32)
    e_r = dot(oh, r2e_ref[...])

    x1 = jnp.maximum(dot(euv, w1a_ref[...]) + dot(e_r, w1b_ref[...])
                     + b1_ref[...], 0.0)
    o = jnp.maximum(dot(x1, w2_ref[...]) + b2_ref[...], 0.0)

    rows_seg = lax.broadcasted_iota(jnp.int32, (_NB, _BB), 0) // L
    Et = (rows_seg == lax.broadcasted_iota(jnp.int32, (_NB, _BB), 1)).astype(f32)
    cols_seg = lax.broadcasted_iota(jnp.int32, (_BB, _NB), 1) // L
    E = (cols_seg == lax.broadcasted_iota(jnp.int32, (_BB, _NB), 0)).astype(f32)

    u_att = dot(urep_ref[...], a1b_ref[...])
    u_exp = dot(Et, u_att)

    a1 = jnp.maximum(dot(o, a1a_ref[...]) + u_exp + ba1_ref[...], 0.0)
    a2 = jnp.maximum(dot(a1, a2w_ref[...]) + ba2_ref[...], 0.0)
    lg = dot(a2, a3w_ref[...])

    el = jnp.exp(lg)
    den = dot(E, el)
    den_exp = dot(Et, den)
    att = el / den_exp
    out_ref[...] = dot(E, o * att)


def _tc_call(euv, hist_r2d, urep, r2e, w1a, w1b, b1, w2, b2,
             a1a, a1b, ba1, a2w, ba2, a3w):
    grid = (B // _BB,)
    full = lambda shape: pl.BlockSpec(shape, lambda i: (0, 0))
    return pl.pallas_call(
        _tc_body,
        grid=grid,
        in_specs=[
            pl.BlockSpec((_NB, D), lambda i: (i, 0)),
            pl.BlockSpec((_NB, 1), lambda i: (i, 0)),
            pl.BlockSpec((_BB, D), lambda i: (i, 0)),
            full((NR, D)),
            full((D, D)), full((D, D)), full((1, D)),
            full((D, D)), full((1, D)),
            full((D, D)), full((D, D)), full((1, D)),
            full((D, D)), full((1, D)), full((D, 1)),
        ],
        out_specs=pl.BlockSpec((_BB, D), lambda i: (i, 0)),
        out_shape=jax.ShapeDtypeStruct((B, D), jnp.float32),
    )(euv, hist_r2d, urep, r2e, w1a, w1b, b1, w2, b2,
      a1a, a1b, ba1, a2w, ba2, a3w)


def kernel(nodes, history_uv, history_r, v2e, u2e, r2e,
           w_r1_W, w_r1_b, w_r2_W, w_r2_b,
           att1_W, att1_b, att2_W, att2_b, att3_W, att3_b):
    vidx = history_uv.reshape(N).astype(jnp.int32)
    nodes32 = nodes.astype(jnp.int32)
    euv, urep = _sc_gather(v2e, vidx, u2e, nodes32)

    hist_r2d = history_r.reshape(N, 1).astype(jnp.int32)
    return _tc_call(
        euv, hist_r2d, urep, r2e,
        w_r1_W[:D], w_r1_W[D:], w_r1_b.reshape(1, D),
        w_r2_W, w_r2_b.reshape(1, D),
        att1_W[:D], att1_W[D:], att1_b.reshape(1, D),
        att2_W, att2_b.reshape(1, D), att3_W,
    )


# speedup vs baseline: 9.4089x; 9.4089x over previous
import functools

import jax
import jax.numpy as jnp
from jax import lax
from jax.experimental import pallas as pl
from jax.experimental.pallas import tpu as pltpu
from jax.experimental.pallas import tpu_sc as plsc

B = 16384
L = 50
D = 16
NR = 5
N = B * L

_NC = 2
_NS = 16
_NW = _NC * _NS
_V_PER_W = N // _NW
_U_PER_W = B // _NW
_CHUNK = 2560
_NSTEP = _V_PER_W // _CHUNK

@functools.cache
def _make_sc_gather():
    mesh = plsc.VectorSubcoreMesh(core_axis_name="c", subcore_axis_name="s",
                                  num_cores=_NC, num_subcores=_NS)

    @functools.partial(
        pl.kernel,
        out_type=[
            jax.ShapeDtypeStruct((N, D), jnp.float32),
            jax.ShapeDtypeStruct((B, D), jnp.float32),
        ],
        mesh=mesh,
        compiler_params=pltpu.CompilerParams(use_tc_tiling_on_sc=False),
        scratch_types=[
            pltpu.VMEM((_CHUNK,), jnp.int32),
            pltpu.VMEM((_CHUNK, D), jnp.float32),
            pltpu.VMEM((_U_PER_W,), jnp.int32),
            pltpu.VMEM((_U_PER_W, D), jnp.float32),
            pltpu.SemaphoreType.DMA,
        ],
    )
    def _sc_gather(v2e_hbm, vidx_hbm, u2e_hbm, nodes_hbm,
                   euv_hbm, urep_hbm,
                   idx_v, rows_v, uidx_v, urows_v, sem):
        wid = lax.axis_index("s") * _NC + lax.axis_index("c")

        ubase = pl.multiple_of(wid * _U_PER_W, 8)
        pltpu.sync_copy(nodes_hbm.at[pl.ds(ubase, _U_PER_W)], uidx_v)
        pltpu.async_copy(u2e_hbm.at[uidx_v], urows_v, sem).wait()
        pltpu.sync_copy(urows_v, urep_hbm.at[pl.ds(ubase, _U_PER_W)])

        vbase = wid * _V_PER_W

        def step(i, carry):
            base = pl.multiple_of(vbase + i * _CHUNK, 8)
            pltpu.sync_copy(vidx_hbm.at[pl.ds(base, _CHUNK)], idx_v)
            pltpu.async_copy(v2e_hbm.at[idx_v], rows_v, sem).wait()
            pltpu.sync_copy(rows_v, euv_hbm.at[pl.ds(base, _CHUNK)])
            return carry

        lax.fori_loop(0, _NSTEP, step, 0)

    return _sc_gather


_BB = 64
_NB = _BB * L


def _tc_body(euv_ref, hr_ref, urep_ref, r2e_ref,
             w1a_ref, w1b_ref, b1_ref, w2_ref, b2_ref,
             a1a_ref, a1b_ref, ba1_ref, a2w_ref, ba2_ref, a3w_ref,
             out_ref):
    f32 = jnp.float32
    dot = functools.partial(jnp.dot, preferred_element_type=f32)

    euv = euv_ref[...]
    hr = hr_ref[...]
    oh = (hr == lax.broadcasted_iota(jnp.int32, (_NB, NR), 1)).astype(f32)
    e_r = dot(oh, r2e_ref[...])

    x1 = jnp.maximum(dot(euv, w1a_ref[...]) + dot(e_r, w1b_ref[...])
                     + b1_ref[...], 0.0)
    o = jnp.maximum(dot(x1, w2_ref[...]) + b2_ref[...], 0.0)

    rows_seg = lax.broadcasted_iota(jnp.int32, (_NB, _BB), 0) // L
    Et = (rows_seg == lax.broadcasted_iota(jnp.int32, (_NB, _BB), 1)).astype(f32)
    cols_seg = lax.broadcasted_iota(jnp.int32, (_BB, _NB), 1) // L
    E = (cols_seg == lax.broadcasted_iota(jnp.int32, (_BB, _NB), 0)).astype(f32)

    u_att = dot(urep_ref[...], a1b_ref[...])
    u_exp = dot(Et, u_att)

    a1 = jnp.maximum(dot(o, a1a_ref[...]) + u_exp + ba1_ref[...], 0.0)
    a2 = jnp.maximum(dot(a1, a2w_ref[...]) + ba2_ref[...], 0.0)
    lg = dot(a2, a3w_ref[...])

    el = jnp.exp(lg)
    den = dot(E, el)
    den_exp = dot(Et, den)
    att = el / den_exp
    out_ref[...] = dot(E, o * att)


def _tc_call(euv, hist_r2d, urep, r2e, w1a, w1b, b1, w2, b2,
             a1a, a1b, ba1, a2w, ba2, a3w):
    grid = (B // _BB,)
    full = lambda shape: pl.BlockSpec(shape, lambda i: (0, 0))
    return pl.pallas_call(
        _tc_body,
        grid=grid,
        in_specs=[
            pl.BlockSpec((_NB, D), lambda i: (i, 0)),
            pl.BlockSpec((_NB, 1), lambda i: (i, 0)),
            pl.BlockSpec((_BB, D), lambda i: (i, 0)),
            full((NR, D)),
            full((D, D)), full((D, D)), full((1, D)),
            full((D, D)), full((1, D)),
            full((D, D)), full((D, D)), full((1, D)),
            full((D, D)), full((1, D)), full((D, 1)),
        ],
        out_specs=pl.BlockSpec((_BB, D), lambda i: (i, 0)),
        out_shape=jax.ShapeDtypeStruct((B, D), jnp.float32),
    )(euv, hist_r2d, urep, r2e, w1a, w1b, b1, w2, b2,
      a1a, a1b, ba1, a2w, ba2, a3w)


def kernel(nodes, history_uv, history_r, v2e, u2e, r2e,
           w_r1_W, w_r1_b, w_r2_W, w_r2_b,
           att1_W, att1_b, att2_W, att2_b, att3_W, att3_b):
    vidx = history_uv.reshape(N).astype(jnp.int32)
    nodes32 = nodes.astype(jnp.int32)
    euv, urep = _make_sc_gather()(v2e, vidx, u2e, nodes32)

    hist_r2d = history_r.reshape(N, 1).astype(jnp.int32)
    return _tc_call(
        euv, hist_r2d, urep, r2e,
        w_r1_W[:D], w_r1_W[D:], w_r1_b.reshape(1, D),
        w_r2_W, w_r2_b.reshape(1, D),
        att1_W[:D], att1_W[D:], att1_b.reshape(1, D),
        att2_W, att2_b.reshape(1, D), att3_W,
    )


# speedup vs baseline: 18.3394x; 1.9491x over previous
import functools

import jax
import jax.numpy as jnp
import numpy as np
from jax import lax
from jax.experimental import pallas as pl
from jax.experimental.pallas import tpu as pltpu
from jax.experimental.pallas import tpu_sc as plsc

B = 16384
L = 50
D = 16
NR = 5
NV = 1000000
NU = 1000000
N = B * L
S = 8
PR = N // S
W128 = S * D

_NC = 2
_NS = 16
_NW = _NC * _NS
_V_PER_W = N // _NW
_U_PER_W = B // _NW
_CHUNK = 2560
_NSTEP = _V_PER_W // _CHUNK


@functools.cache
def _make_sc_gather():
    mesh = plsc.VectorSubcoreMesh(core_axis_name="c", subcore_axis_name="s",
                                  num_cores=_NC, num_subcores=_NS)

    @functools.partial(
        pl.kernel,
        out_type=[
            jax.ShapeDtypeStruct((N, D), jnp.float32),
            jax.ShapeDtypeStruct((B, D), jnp.float32),
        ],
        mesh=mesh,
        compiler_params=pltpu.CompilerParams(use_tc_tiling_on_sc=False),
        scratch_types=[
            pltpu.VMEM((_CHUNK,), jnp.int32),
            pltpu.VMEM((_CHUNK, D), jnp.float32),
            pltpu.VMEM((_U_PER_W,), jnp.int32),
            pltpu.VMEM((_U_PER_W, D), jnp.float32),
            pltpu.SemaphoreType.DMA,
        ],
    )
    def _sc_gather(v2e_hbm, vidx_hbm, u2e_hbm, nodes_hbm,
                   euv_hbm, urep_hbm,
                   idx_v, rows_v, uidx_v, urows_v, sem):
        wid = lax.axis_index("s") * _NC + lax.axis_index("c")
        v2e2 = v2e_hbm
        u2e2 = u2e_hbm

        ubase = pl.multiple_of(wid * _U_PER_W, 8)
        pltpu.sync_copy(nodes_hbm.at[pl.ds(ubase, _U_PER_W)], uidx_v)
        pltpu.async_copy(u2e2.at[uidx_v], urows_v, sem).wait()
        pltpu.sync_copy(urows_v, urep_hbm.at[pl.ds(ubase, _U_PER_W)])

        vbase = wid * _V_PER_W

        def step(i, carry):
            base = pl.multiple_of(vbase + i * _CHUNK, 8)
            pltpu.sync_copy(vidx_hbm.at[pl.ds(base, _CHUNK)], idx_v)
            pltpu.async_copy(v2e2.at[idx_v], rows_v, sem).wait()
            pltpu.sync_copy(rows_v, euv_hbm.at[pl.ds(base, _CHUNK)])
            return carry

        lax.fori_loop(0, _NSTEP, step, 0)

    return _sc_gather


_GB = 16
_BBLK = _GB * S
_MB = _GB * L


def _tc_body(euv_ref, hr40_ref, urep_ref,
             c1p_ref, w1ap_ref, w2p_ref, b2p_ref,
             a1ap_ref, a1bp_ref, ba1p_ref, a2p_ref, ba2p_ref,
             a3p_ref, r8_ref, out_ref):
    f32 = jnp.float32
    dot = functools.partial(jnp.dot, preferred_element_type=f32)

    euv = euv_ref[...]
    hr40 = hr40_ref[...]
    ohp = (hr40 == lax.broadcasted_iota(jnp.int32, (_MB, S * NR), 1) % NR
           ).astype(f32)

    x1 = jnp.maximum(dot(euv, w1ap_ref[...]) + dot(ohp, c1p_ref[...]), 0.0)
    o = jnp.maximum(dot(x1, w2p_ref[...]) + b2p_ref[...], 0.0)

    etg = (lax.broadcasted_iota(jnp.int32, (_MB, _GB), 0) // L ==
           lax.broadcasted_iota(jnp.int32, (_MB, _GB), 1)).astype(f32)
    eg = (lax.broadcasted_iota(jnp.int32, (_GB, _MB), 0) ==
          lax.broadcasted_iota(jnp.int32, (_GB, _MB), 1) // L).astype(f32)

    u_att = dot(urep_ref[...], a1bp_ref[...]) + ba1p_ref[...]
    u_exp = dot(etg, u_att)

    a1 = jnp.maximum(dot(o, a1ap_ref[...]) + u_exp, 0.0)
    a2 = jnp.maximum(dot(a1, a2p_ref[...]) + ba2p_ref[...], 0.0)
    lg = dot(a2, a3p_ref[...])

    el = jnp.exp(lg)
    den = dot(eg, el)
    dexp = dot(etg, 1.0 / den)
    att = el * dexp
    att128 = dot(att, r8_ref[...])
    out_ref[...] = dot(eg, o * att128)


def _tc_call(euv128, hr40, urep128, c1p, w1ap, w2p, b2p,
             a1ap, a1bp, ba1p, a2p, ba2p, a3p, r8):
    grid = (B // _BBLK,)
    full = lambda shape: pl.BlockSpec(shape, lambda i: (0, 0))
    return pl.pallas_call(
        _tc_body,
        grid=grid,
        in_specs=[
            pl.BlockSpec((_MB, W128), lambda i: (i, 0)),
            pl.BlockSpec((_MB, S * NR), lambda i: (i, 0)),
            pl.BlockSpec((_GB, W128), lambda i: (i, 0)),
            full((S * NR, W128)),
            full((W128, W128)), full((W128, W128)), full((1, W128)),
            full((W128, W128)), full((W128, W128)), full((1, W128)),
            full((W128, W128)), full((1, W128)),
            full((W128, S)), full((S, W128)),
        ],
        out_specs=pl.BlockSpec((_GB, W128), lambda i: (i, 0)),
        out_shape=jax.ShapeDtypeStruct((B // S, W128), jnp.float32),
        compiler_params=pltpu.CompilerParams(
            dimension_semantics=("parallel",)),
    )(euv128, hr40, urep128, c1p, w1ap, w2p, b2p,
      a1ap, a1bp, ba1p, a2p, ba2p, a3p, r8)


def kernel(nodes, history_uv, history_r, v2e, u2e, r2e,
           w_r1_W, w_r1_b, w_r2_W, w_r2_b,
           att1_W, att1_b, att2_W, att2_b, att3_W, att3_b):
    f32 = jnp.float32
    nblk = B // _BBLK

    perm4 = lambda a: a.reshape(nblk, _GB, S, L).transpose(0, 1, 3, 2)
    vidx = perm4(history_uv).reshape(N).astype(jnp.int32)
    hrp = perm4(history_r).reshape(PR, S).astype(jnp.int32)
    hr40 = jnp.repeat(hrp, NR, axis=1)

    euv, urep = _make_sc_gather()(
        v2e, vidx, u2e, nodes.astype(jnp.int32))
    euv128 = euv.reshape(PR, W128)
    urep128 = urep.reshape(B // S, W128)

    eye8 = jnp.eye(S, dtype=f32)
    kron = lambda w: jnp.kron(eye8, w.astype(f32))
    tile8 = lambda b: jnp.tile(b.reshape(1, -1), (1, S))

    c1 = r2e @ w_r1_W[D:] + w_r1_b
    out128 = _tc_call(
        euv128, hr40, urep128,
        kron(c1), kron(w_r1_W[:D]),
        kron(w_r2_W), tile8(w_r2_b),
        kron(att1_W[:D]), kron(att1_W[D:]), tile8(att1_b),
        kron(att2_W), tile8(att2_b),
        kron(att3_W),
        kron(jnp.ones((1, D), f32)),
    )
    return out128.reshape(B, D)
